# baseline (speedup 1.0000x reference)
import functools

import jax
import jax.numpy as jnp
from jax import lax
from jax.experimental import pallas as pl
from jax.experimental.pallas import tpu as pltpu
from jax.experimental.pallas import tpu_sc as plsc

BATCH = 16384
HIST = 50
DIM = 64
NUM_EMB = 1000000
NUM_WORKERS = 32
BAGS_PER_WORKER = BATCH // NUM_WORKERS
CHUNK_BAGS = 16
ROWS_PER_CHUNK = CHUNK_BAGS * HIST
NUM_CHUNKS = BAGS_PER_WORKER // CHUNK_BAGS
IDX_COPY = 896
LANES = 16
DSUB = DIM // LANES


def _sc_bag_mean(flat_idx, weight):
    mesh = plsc.VectorSubcoreMesh(core_axis_name="c", subcore_axis_name="s")

    @functools.partial(
        pl.kernel,
        mesh=mesh,
        compiler_params=pltpu.CompilerParams(use_tc_tiling_on_sc=False),
        out_type=jax.ShapeDtypeStruct((BATCH, DIM), jnp.float32),
        scratch_types=[
            pltpu.VMEM((IDX_COPY,), jnp.int32),
            pltpu.VMEM((IDX_COPY,), jnp.int32),
            pltpu.VMEM((ROWS_PER_CHUNK, DIM), jnp.float32),
            pltpu.VMEM((ROWS_PER_CHUNK, DIM), jnp.float32),
            pltpu.VMEM((CHUNK_BAGS, DIM), jnp.float32),
            pltpu.SemaphoreType.DMA,
            pltpu.SemaphoreType.DMA,
        ],
    )
    def k(idx_hbm, w_hbm, out_hbm, idx0_v, idx1_v, rows0_v, rows1_v,
          acc_v, sem0, sem1):
        wid = lax.axis_index("s") * 2 + lax.axis_index("c")
        bag_base = wid * BAGS_PER_WORKER

        def start_gather(chunk, idx_v, rows_v, sem):
            first_bag = bag_base + chunk * CHUNK_BAGS
            pltpu.sync_copy(
                idx_hbm.at[pl.ds(first_bag * HIST, IDX_COPY)], idx_v)
            pltpu.async_copy(
                w_hbm.at[idx_v.at[pl.ds(0, ROWS_PER_CHUNK)]], rows_v, sem)

        def finish_chunk(chunk, idx_v, rows_v, sem):
            pltpu.make_async_copy(
                w_hbm.at[idx_v.at[pl.ds(0, ROWS_PER_CHUNK)]], rows_v,
                sem).wait()

            def bag_body(c, _):
                base_row = c * HIST
                accs = [jnp.zeros((LANES,), jnp.float32) for _ in range(DSUB)]
                for r in range(HIST):
                    for j in range(DSUB):
                        accs[j] = accs[j] + rows_v[base_row + r,
                                                   pl.ds(j * LANES, LANES)]
                for j in range(DSUB):
                    acc_v[c, pl.ds(j * LANES, LANES)] = accs[j]
                return ()

            lax.fori_loop(0, CHUNK_BAGS, bag_body, ())
            first_bag = bag_base + chunk * CHUNK_BAGS
            pltpu.sync_copy(acc_v, out_hbm.at[pl.ds(first_bag, CHUNK_BAGS)])

        start_gather(0, idx0_v, rows0_v, sem0)

        def pair_body(p, _):
            c0 = 2 * p
            start_gather(c0 + 1, idx1_v, rows1_v, sem1)
            finish_chunk(c0, idx0_v, rows0_v, sem0)

            @pl.when(p < NUM_CHUNKS // 2 - 1)
            def _():
                start_gather(c0 + 2, idx0_v, rows0_v, sem0)

            finish_chunk(c0 + 1, idx1_v, rows1_v, sem1)
            return ()

        lax.fori_loop(0, NUM_CHUNKS // 2, pair_body, ())

    return k(flat_idx, weight)


def kernel(indices, weight):
    flat_idx = indices.reshape(-1).astype(jnp.int32)
    flat_idx = jnp.concatenate(
        [flat_idx, jnp.zeros((IDX_COPY - ROWS_PER_CHUNK,), jnp.int32)])
    w_flat = lax.optimization_barrier(
        (weight * jnp.float32(1.0 / HIST)).reshape(-1))
    return _sc_bag_mean(flat_idx, w_flat.reshape(NUM_EMB, DIM))


# speedup vs baseline: 1.2058x; 1.2058x over previous
import functools

import jax
import jax.numpy as jnp
from jax import lax
from jax.experimental import pallas as pl
from jax.experimental.pallas import tpu as pltpu
from jax.experimental.pallas import tpu_sc as plsc

BATCH = 16384
HIST = 50
DIM = 64
NUM_EMB = 1000000
NUM_WORKERS = 32
BAGS_PER_WORKER = BATCH // NUM_WORKERS
CHUNK_BAGS = 16
ROWS_PER_CHUNK = CHUNK_BAGS * HIST
NUM_CHUNKS = BAGS_PER_WORKER // CHUNK_BAGS
IDX_COPY = 896
LANES = 16
DSUB = DIM // LANES


def _sc_bag_mean(flat_idx, weight):
    mesh = plsc.VectorSubcoreMesh(core_axis_name="c", subcore_axis_name="s")

    @functools.partial(
        pl.kernel,
        mesh=mesh,
        compiler_params=pltpu.CompilerParams(use_tc_tiling_on_sc=False),
        out_type=jax.ShapeDtypeStruct((BATCH, DIM), jnp.float32),
        scratch_types=[
            pltpu.VMEM((IDX_COPY,), jnp.int32),
            pltpu.VMEM((IDX_COPY,), jnp.int32),
            pltpu.VMEM((ROWS_PER_CHUNK, DIM), jnp.float32),
            pltpu.VMEM((ROWS_PER_CHUNK, DIM), jnp.float32),
            pltpu.VMEM((CHUNK_BAGS, DIM), jnp.float32),
            pltpu.SemaphoreType.DMA,
            pltpu.SemaphoreType.DMA,
        ],
    )
    def k(idx_hbm, w_hbm, out_hbm, idx0_v, idx1_v, rows0_v, rows1_v,
          acc_v, sem0, sem1):
        wid = lax.axis_index("s") * 2 + lax.axis_index("c")
        bag_base = wid * BAGS_PER_WORKER
        scale = jnp.full((LANES,), 1.0 / HIST, jnp.float32)

        def start_gather(chunk, idx_v, rows_v, sem):
            first_bag = bag_base + chunk * CHUNK_BAGS
            pltpu.sync_copy(
                idx_hbm.at[pl.ds(first_bag * HIST, IDX_COPY)], idx_v)
            pltpu.async_copy(
                w_hbm.at[idx_v.at[pl.ds(0, ROWS_PER_CHUNK)]], rows_v, sem)

        def finish_chunk(chunk, idx_v, rows_v, sem):
            pltpu.make_async_copy(
                w_hbm.at[idx_v.at[pl.ds(0, ROWS_PER_CHUNK)]], rows_v,
                sem).wait()

            def bag_body(c, _):
                base_row = c * HIST
                accs = [jnp.zeros((LANES,), jnp.float32) for _ in range(DSUB)]
                for r in range(HIST):
                    for j in range(DSUB):
                        accs[j] = accs[j] + rows_v[base_row + r,
                                                   pl.ds(j * LANES, LANES)]
                for j in range(DSUB):
                    acc_v[c, pl.ds(j * LANES, LANES)] = accs[j] * scale
                return ()

            lax.fori_loop(0, CHUNK_BAGS, bag_body, ())
            first_bag = bag_base + chunk * CHUNK_BAGS
            pltpu.sync_copy(acc_v, out_hbm.at[pl.ds(first_bag, CHUNK_BAGS)])

        start_gather(0, idx0_v, rows0_v, sem0)

        def pair_body(p, _):
            c0 = 2 * p
            start_gather(c0 + 1, idx1_v, rows1_v, sem1)
            finish_chunk(c0, idx0_v, rows0_v, sem0)

            @pl.when(p < NUM_CHUNKS // 2 - 1)
            def _():
                start_gather(c0 + 2, idx0_v, rows0_v, sem0)

            finish_chunk(c0 + 1, idx1_v, rows1_v, sem1)
            return ()

        lax.fori_loop(0, NUM_CHUNKS // 2, pair_body, ())

    return k(flat_idx, weight)


def kernel(indices, weight):
    flat_idx = indices.reshape(-1).astype(jnp.int32)
    flat_idx = jnp.concatenate(
        [flat_idx, jnp.zeros((IDX_COPY - ROWS_PER_CHUNK,), jnp.int32)])
    return _sc_bag_mean(flat_idx, weight)
